# baseline (speedup 1.0000x reference)
import jax
import jax.numpy as jnp
from jax import lax
from jax.experimental import pallas as pl
from jax.experimental.pallas import tpu as pltpu
from jax.experimental.pallas import tpu_sc as plsc

VOCAB = 1000000
D = 64
B = 16384
NEG = 5

NC = 2
NS = 16
NW = NC * NS
L = 16
BPW = B // NW
CH = 64
NR = BPW // CH
GPR = CH // L
NIC = BPW // 128

TBLK = 4096
HALF = 1 << 19
HB = HALF // TBLK


def _wbuild_body(ul_ref, uh_ref, vl_ref, vh_ref, e_ref, o_ref):
    dn = (((0,), (0,)), ((), ()))
    e = e_ref[...]

    def tr2(a_ref, b_ref):
        z = jnp.concatenate([a_ref[...], b_ref[...]], axis=0)
        return lax.dot_general(z, e, dn,
                               preferred_element_type=jnp.float32)

    def pack(y):
        yu = lax.bitcast_convert_type(y, jnp.uint32)
        lo = yu[:, 0:D] >> 16
        hi = yu[:, D:2 * D] & jnp.uint32(0xFFFF0000)
        return lax.bitcast_convert_type(lo | hi, jnp.int32)

    o_ref[:, 0:D] = pack(tr2(ul_ref, vl_ref))
    o_ref[:, D:2 * D] = pack(tr2(uh_ref, vh_ref))


RPR = (2 + NEG) * CH


def _sc_body(m3q, m3cb, W_hbm, out_hbm, midx, mcb, rows_v, sbuf, sem0, sem1):
    wid = lax.axis_index("s") * NC + lax.axis_index("c")

    pltpu.sync_copy(m3q.at[wid], midx)
    pltpu.sync_copy(m3cb.at[wid], mcb)

    lane = lax.iota(jnp.int32, L)
    sems = (sem0, sem1)

    def fire(r):
        b = r % 2
        s = sems[b]
        cps = []
        for (o, n) in ((0, 128), (128, 128), (256, 128), (384, 64)):
            cps.append(pltpu.async_copy(
                W_hbm.at[midx.at[r, pl.ds(o, n)]],
                rows_v.at[b, pl.ds(o, n)], s))
        return cps

    pend = fire(0)
    for r in range(NR):
        nxt = fire(r + 1) if r + 1 < NR else []
        for c in pend:
            c.wait()
        pend = nxt
        b = r % 2
        bvec = jnp.full((L,), b, jnp.int32)

        for go in range(GPR):
            rows = go * L + lane
            cbs = [mcb[r, pl.ds(j * CH + go * L, L)] for j in range(2 + NEG)]

            def d_body(d, carry):
                acc_p, acc_n = carry
                du = jnp.full((L,), d, jnp.int32)
                wu = plsc.load_gather(rows_v, [bvec, rows, cbs[0] + du])
                wv = plsc.load_gather(rows_v, [bvec, rows + CH, cbs[1] + du])
                uf = plsc.bitcast(wu << 16, jnp.float32)
                vf = plsc.bitcast(wv & jnp.int32(-65536), jnp.float32)
                nf = None
                for k in range(NEG):
                    wn = plsc.load_gather(
                        rows_v, [bvec, rows + (2 + k) * CH, cbs[2 + k] + du])
                    x = plsc.bitcast(wn & jnp.int32(-65536), jnp.float32)
                    nf = x if nf is None else nf + x
                return acc_p + uf * vf, acc_n + uf * nf

            z = jnp.zeros((L,), jnp.float32)
            acc_p, acc_n = lax.fori_loop(0, D, d_body, (z, z), unroll=4)
            off = (r & 1) * CH + go * L
            sbuf[r >> 1, pl.ds(off, L)] = acc_p
            sbuf[NIC + (r >> 1), pl.ds(off, L)] = acc_n

    pltpu.sync_copy(sbuf, out_hbm.at[wid])


def _loss_body(x_ref, o_ref):
    s = x_ref[:, 0:NIC, :]
    n = -x_ref[:, NIC:2 * NIC, :]

    def ls(x):
        return jnp.minimum(x, 0.0) - jnp.log1p(jnp.exp(-jnp.abs(x)))

    o_ref[...] = (-(jnp.sum(ls(s) + ls(n))) / B).reshape(1, 1)


def kernel(u, v, neg_v, U, V):
    eye = jnp.eye(2 * D, dtype=jnp.float32)
    W32 = pl.pallas_call(
        _wbuild_body,
        grid=(HB,),
        in_specs=[
            pl.BlockSpec((D, TBLK), lambda j: (0, j)),
            pl.BlockSpec((D, TBLK), lambda j: (0, jnp.minimum(j + HB, VOCAB // TBLK))),
            pl.BlockSpec((D, TBLK), lambda j: (0, j)),
            pl.BlockSpec((D, TBLK), lambda j: (0, jnp.minimum(j + HB, VOCAB // TBLK))),
            pl.BlockSpec((2 * D, 2 * D), lambda j: (0, 0)),
        ],
        out_specs=pl.BlockSpec((TBLK, 2 * D), lambda j: (j, 0)),
        out_shape=jax.ShapeDtypeStruct((HALF, 2 * D), jnp.int32),
    )(U.T, U.T, V.T, V.T, eye)

    m3 = jnp.concatenate(
        [u.astype(jnp.int32).reshape(NW, NR, CH),
         v.astype(jnp.int32).reshape(NW, NR, CH),
         neg_v.astype(jnp.int32).T.reshape(NEG, NW, NR, CH)
         .transpose(1, 2, 0, 3).reshape(NW, NR, NEG * CH)],
        axis=2)
    m3q = m3 & (HALF - 1)
    m3cb = (m3 >> 19) << 6

    mesh = plsc.VectorSubcoreMesh(core_axis_name="c", subcore_axis_name="s")
    packed = pl.kernel(
        _sc_body,
        out_type=jax.ShapeDtypeStruct((NW, 2 * NIC, 128), jnp.float32),
        mesh=mesh,
        compiler_params=pltpu.CompilerParams(needs_layout_passes=False),
        scratch_types=[
            pltpu.VMEM((NR, RPR), jnp.int32),
            pltpu.VMEM((NR, RPR), jnp.int32),
            pltpu.VMEM((2, RPR, 2 * D), jnp.int32),
            pltpu.VMEM((2 * NIC, 128), jnp.float32),
            pltpu.SemaphoreType.DMA,
            pltpu.SemaphoreType.DMA,
        ],
    )(m3q, m3cb, W32)

    loss = pl.pallas_call(
        _loss_body,
        out_shape=jax.ShapeDtypeStruct((1, 1), jnp.float32),
    )(packed)
    return loss[0, 0]


# speedup vs baseline: 1.0763x; 1.0763x over previous
import jax
import jax.numpy as jnp
from jax import lax
from jax.experimental import pallas as pl
from jax.experimental.pallas import tpu as pltpu
from jax.experimental.pallas import tpu_sc as plsc

VOCAB = 1000000
D = 64
B = 16384
NEG = 5

NC = 2
NS = 16
NW = NC * NS
L = 16
BPW = B // NW
CH = 64
NR = BPW // CH
GPR = CH // L
NIC = BPW // 128

TBLK = 8192
HALF = 1 << 19
HB = HALF // TBLK


def _wbuild_body(ul_ref, uh_ref, vl_ref, vh_ref, e_ref, o_ref):
    dn = (((0,), (0,)), ((), ()))
    e = e_ref[...]

    def tr2(a_ref, b_ref):
        z = jnp.concatenate([a_ref[...], b_ref[...]], axis=0)
        return lax.dot_general(z, e, dn,
                               preferred_element_type=jnp.float32)

    def pack(y):
        yu = lax.bitcast_convert_type(y, jnp.uint32)
        lo = yu[:, 0:D] >> 16
        hi = yu[:, D:2 * D] & jnp.uint32(0xFFFF0000)
        return lax.bitcast_convert_type(lo | hi, jnp.int32)

    o_ref[:, 0:D] = pack(tr2(ul_ref, vl_ref))
    o_ref[:, D:2 * D] = pack(tr2(uh_ref, vh_ref))


RPR = (2 + NEG) * CH


def _sc_body(m3q, m3cb, W_hbm, out_hbm, midx, mcb, rows_v, sbuf, sem0, sem1):
    wid = lax.axis_index("s") * NC + lax.axis_index("c")

    pltpu.sync_copy(m3q.at[wid], midx)
    pltpu.sync_copy(m3cb.at[wid], mcb)

    lane = lax.iota(jnp.int32, L)
    sems = (sem0, sem1)

    def fire(r):
        b = r % 2
        s = sems[b]
        cps = []
        for (o, n) in ((0, 128), (128, 128), (256, 128), (384, 64)):
            cps.append(pltpu.async_copy(
                W_hbm.at[midx.at[r, pl.ds(o, n)]],
                rows_v.at[b, pl.ds(o, n)], s))
        return cps

    pend = fire(0)
    for r in range(NR):
        nxt = fire(r + 1) if r + 1 < NR else []
        for c in pend:
            c.wait()
        pend = nxt
        b = r % 2
        bvec = jnp.full((L,), b, jnp.int32)

        for go in range(GPR):
            rows = go * L + lane
            cbs = [mcb[r, pl.ds(j * CH + go * L, L)] for j in range(2 + NEG)]

            def d_body(d, carry):
                acc_p, acc_n = carry
                du = jnp.full((L,), d, jnp.int32)
                wu = plsc.load_gather(rows_v, [bvec, rows, cbs[0] + du])
                wv = plsc.load_gather(rows_v, [bvec, rows + CH, cbs[1] + du])
                uf = plsc.bitcast(wu << 16, jnp.float32)
                vf = plsc.bitcast(wv & jnp.int32(-65536), jnp.float32)
                nf = None
                for k in range(NEG):
                    wn = plsc.load_gather(
                        rows_v, [bvec, rows + (2 + k) * CH, cbs[2 + k] + du])
                    x = plsc.bitcast(wn & jnp.int32(-65536), jnp.float32)
                    nf = x if nf is None else nf + x
                return acc_p + uf * vf, acc_n + uf * nf

            z = jnp.zeros((L,), jnp.float32)
            acc_p, acc_n = lax.fori_loop(0, D, d_body, (z, z), unroll=4)
            off = (r & 1) * CH + go * L
            sbuf[r >> 1, pl.ds(off, L)] = acc_p
            sbuf[NIC + (r >> 1), pl.ds(off, L)] = acc_n

    pltpu.sync_copy(sbuf, out_hbm.at[wid])


def _loss_body(x_ref, o_ref):
    s = x_ref[:, 0:NIC, :]
    n = -x_ref[:, NIC:2 * NIC, :]

    def ls(x):
        return jnp.minimum(x, 0.0) - jnp.log1p(jnp.exp(-jnp.abs(x)))

    o_ref[...] = (-(jnp.sum(ls(s) + ls(n))) / B).reshape(1, 1)


def kernel(u, v, neg_v, U, V):
    eye = jnp.eye(2 * D, dtype=jnp.float32)
    W32 = pl.pallas_call(
        _wbuild_body,
        grid=(HB,),
        in_specs=[
            pl.BlockSpec((D, TBLK), lambda j: (0, j)),
            pl.BlockSpec((D, TBLK), lambda j: (0, jnp.minimum(j + HB, VOCAB // TBLK))),
            pl.BlockSpec((D, TBLK), lambda j: (0, j)),
            pl.BlockSpec((D, TBLK), lambda j: (0, jnp.minimum(j + HB, VOCAB // TBLK))),
            pl.BlockSpec((2 * D, 2 * D), lambda j: (0, 0)),
        ],
        out_specs=pl.BlockSpec((TBLK, 2 * D), lambda j: (j, 0)),
        out_shape=jax.ShapeDtypeStruct((HALF, 2 * D), jnp.int32),
    )(U.T, U.T, V.T, V.T, eye)

    m3 = jnp.concatenate(
        [u.astype(jnp.int32).reshape(NW, NR, CH),
         v.astype(jnp.int32).reshape(NW, NR, CH),
         neg_v.astype(jnp.int32).T.reshape(NEG, NW, NR, CH)
         .transpose(1, 2, 0, 3).reshape(NW, NR, NEG * CH)],
        axis=2)
    m3q = m3 & (HALF - 1)
    m3cb = (m3 >> 19) << 6

    mesh = plsc.VectorSubcoreMesh(core_axis_name="c", subcore_axis_name="s")
    packed = pl.kernel(
        _sc_body,
        out_type=jax.ShapeDtypeStruct((NW, 2 * NIC, 128), jnp.float32),
        mesh=mesh,
        compiler_params=pltpu.CompilerParams(needs_layout_passes=False),
        scratch_types=[
            pltpu.VMEM((NR, RPR), jnp.int32),
            pltpu.VMEM((NR, RPR), jnp.int32),
            pltpu.VMEM((2, RPR, 2 * D), jnp.int32),
            pltpu.VMEM((2 * NIC, 128), jnp.float32),
            pltpu.SemaphoreType.DMA,
            pltpu.SemaphoreType.DMA,
        ],
    )(m3q, m3cb, W32)

    loss = pl.pallas_call(
        _loss_body,
        out_shape=jax.ShapeDtypeStruct((1, 1), jnp.float32),
    )(packed)
    return loss[0, 0]
